# Initial kernel scaffold; baseline (speedup 1.0000x reference)
import jax
import jax.numpy as jnp
from jax.experimental import pallas as pl


def kernel(users, movies, user_emb, movie_emb, user_bias_t, movie_bias_t, global_bias, gu, bu, gm, bm, W1, b1, g1, be1, W2, b2, g2, be2, W3, b3, g3, be3, Wf, bf, Wr, br):
    raise NotImplementedError("write your pallas kernel here")


# speedup vs baseline: 1.1865x; 1.1865x over previous
import functools

import jax
import jax.numpy as jnp
from jax import lax
from jax.experimental import pallas as pl
from jax.experimental.pallas import tpu as pltpu
from jax.experimental.pallas import tpu_sc as plsc

_B = 16384
_D = 128
_NC = 2
_NS = 16
_NW = _NC * _NS
_BPW = _B // _NW
_CH = _BPW // 128
_EPS = 1e-5

_BLK = 1024
_NBLK = _B // _BLK


def _sc_gather_body(uidx_hbm, midx_hbm, uemb_hbm, memb_hbm, ubias_hbm,
                    mbias_hbm, ue_out, me_out, ub_out, mb_out,
                    idx_u, idx_m, rows, brows_u, brows_m, sem_e, sem_b):
    wid = lax.axis_index("s") * _NC + lax.axis_index("c")
    base = wid * _BPW
    pltpu.sync_copy(uidx_hbm.at[wid], idx_u)
    pltpu.sync_copy(midx_hbm.at[wid], idx_m)
    hb = []
    for j in range(_CH):
        hb.append(pltpu.async_copy(ubias_hbm.at[idx_u.at[j]],
                                   brows_u.at[pl.ds(j * 128, 128)], sem_b))
        hb.append(pltpu.async_copy(mbias_hbm.at[idx_m.at[j]],
                                   brows_m.at[pl.ds(j * 128, 128)], sem_b))
    he = [pltpu.async_copy(uemb_hbm.at[idx_u.at[j]],
                           rows.at[pl.ds(j * 128, 128)], sem_e)
          for j in range(_CH)]
    for h in he:
        h.wait()
    pltpu.sync_copy(rows, ue_out.at[pl.ds(base, _BPW)])
    he = [pltpu.async_copy(memb_hbm.at[idx_m.at[j]],
                           rows.at[pl.ds(j * 128, 128)], sem_e)
          for j in range(_CH)]
    for h in hb:
        h.wait()
    pltpu.sync_copy(brows_u, ub_out.at[pl.ds(base, _BPW)])
    pltpu.sync_copy(brows_m, mb_out.at[pl.ds(base, _BPW)])
    for h in he:
        h.wait()
    pltpu.sync_copy(rows, me_out.at[pl.ds(base, _BPW)])


def _sc_gather(uidx, midx, uemb, memb, ubias, mbias):
    mesh = plsc.VectorSubcoreMesh(core_axis_name="c", subcore_axis_name="s")
    fn = pl.kernel(
        _sc_gather_body,
        mesh=mesh,
        out_type=(
            jax.ShapeDtypeStruct((_B, _D), jnp.float32),
            jax.ShapeDtypeStruct((_B, _D), jnp.float32),
            jax.ShapeDtypeStruct((_B,), jnp.float32),
            jax.ShapeDtypeStruct((_B,), jnp.float32),
        ),
        scratch_types=[
            pltpu.VMEM((_CH, 128), jnp.int32),
            pltpu.VMEM((_CH, 128), jnp.int32),
            pltpu.VMEM((_BPW, _D), jnp.float32),
            pltpu.VMEM((_BPW,), jnp.float32),
            pltpu.VMEM((_BPW,), jnp.float32),
            pltpu.SemaphoreType.DMA,
            pltpu.SemaphoreType.DMA,
        ],
    )
    return fn(uidx, midx, uemb, memb, ubias, mbias)


def _p0_body(ue_ref, me_ref, stats_ref):
    i = pl.program_id(0)

    @pl.when(i == 0)
    def _():
        stats_ref[...] = jnp.zeros_like(stats_ref)

    ue = ue_ref[...]
    me = me_ref[...]
    stats_ref[...] += jnp.stack([
        jnp.sum(ue, axis=0), jnp.sum(ue * ue, axis=0),
        jnp.sum(me, axis=0), jnp.sum(me * me, axis=0),
    ])


def _affine(s_sum, s_sq, g, be):
    mu = s_sum * (1.0 / _B)
    var = s_sq * (1.0 / _B) - mu * mu
    a = g * lax.rsqrt(var + _EPS)
    return a, be - mu * a


def _p1_body(ue_ref, me_ref, s0_ref, gu_ref, bu_ref, gm_ref, bm_ref,
             w1t_ref, b1_ref, wr_ref, sc_ref, ub_ref, mb_ref,
             y1_ref, res_ref, s1_ref):
    i = pl.program_id(0)
    s0 = s0_ref[...]
    au, cu = _affine(s0[0:1], s0[1:2], gu_ref[...], bu_ref[...])
    am, cm = _affine(s0[2:3], s0[3:4], gm_ref[...], bm_ref[...])
    comb = jnp.concatenate([ue_ref[...] * au + cu,
                            me_ref[...] * am + cm], axis=1)
    y1 = jnp.dot(comb, w1t_ref[...],
                 preferred_element_type=jnp.float32) + b1_ref[...]
    y1_ref[...] = y1
    res_ref[...] = (jnp.sum(comb * wr_ref[...], axis=1)
                    + sc_ref[0, 0] + ub_ref[...] + mb_ref[...])

    @pl.when(i == 0)
    def _():
        s1_ref[...] = jnp.zeros_like(s1_ref)

    s1_ref[...] += jnp.stack([jnp.sum(y1, axis=0), jnp.sum(y1 * y1, axis=0)])


def _mid_body(y_ref, s_ref, g_ref, be_ref, wt_ref, b_ref, yn_ref, sn_ref):
    i = pl.program_id(0)
    s = s_ref[...]
    a, c = _affine(s[0:1], s[1:2], g_ref[...], be_ref[...])
    x = jnp.maximum(y_ref[...] * a + c, 0.0)
    yn = jnp.dot(x, wt_ref[...],
                 preferred_element_type=jnp.float32) + b_ref[...]
    yn_ref[...] = yn

    @pl.when(i == 0)
    def _():
        sn_ref[...] = jnp.zeros_like(sn_ref)

    sn_ref[...] += jnp.stack([jnp.sum(yn, axis=0), jnp.sum(yn * yn, axis=0)])


def _p4_body(y3_ref, s3_ref, g3_ref, be3_ref, wf_ref, bf_ref, res_ref,
             out_ref):
    s = s3_ref[...]
    a, c = _affine(s[0:1], s[1:2], g3_ref[...], be3_ref[...])
    x = jnp.maximum(y3_ref[...] * a + c, 0.0)
    out_ref[...] = jnp.sum(x * wf_ref[...], axis=1) + bf_ref[0, 0] + res_ref[...]


def _row_spec(f):
    return pl.BlockSpec((_BLK, f), lambda i: (i, 0))


def _vec_spec():
    return pl.BlockSpec((_BLK,), lambda i: (i,))


def _full_spec(shape):
    nd = len(shape)
    return pl.BlockSpec(shape, lambda i: (0,) * nd)


def kernel(users, movies, user_emb, movie_emb, user_bias_t, movie_bias_t,
           global_bias, gu, bu, gm, bm, W1, b1, g1, be1, W2, b2, g2, be2,
           W3, b3, g3, be3, Wf, bf, Wr, br):
    uidx = users.astype(jnp.int32).reshape(_NW, _CH, 128)
    midx = movies.astype(jnp.int32).reshape(_NW, _CH, 128)
    ue, me, ubg, mbg = _sc_gather(uidx, midx, user_emb, movie_emb,
                                  user_bias_t.reshape(-1),
                                  movie_bias_t.reshape(-1))

    f32 = jnp.float32
    stats0 = pl.pallas_call(
        _p0_body,
        grid=(_NBLK,),
        in_specs=[_row_spec(_D), _row_spec(_D)],
        out_specs=_full_spec((4, _D)),
        out_shape=jax.ShapeDtypeStruct((4, _D), f32),
    )(ue, me)

    scalar_c = (global_bias + br).reshape(1, 1)
    y1, res, s1 = pl.pallas_call(
        _p1_body,
        grid=(_NBLK,),
        in_specs=[_row_spec(_D), _row_spec(_D), _full_spec((4, _D)),
                  _full_spec((1, _D)), _full_spec((1, _D)),
                  _full_spec((1, _D)), _full_spec((1, _D)),
                  _full_spec((2 * _D, 512)), _full_spec((1, 512)),
                  _full_spec((1, 2 * _D)), _full_spec((1, 1)),
                  _vec_spec(), _vec_spec()],
        out_specs=[_row_spec(512), _vec_spec(), _full_spec((2, 512))],
        out_shape=[jax.ShapeDtypeStruct((_B, 512), f32),
                   jax.ShapeDtypeStruct((_B,), f32),
                   jax.ShapeDtypeStruct((2, 512), f32)],
    )(ue, me, stats0, gu.reshape(1, -1), bu.reshape(1, -1),
      gm.reshape(1, -1), bm.reshape(1, -1), W1.T, b1.reshape(1, -1),
      Wr, scalar_c, ubg, mbg)

    y2, s2 = pl.pallas_call(
        _mid_body,
        grid=(_NBLK,),
        in_specs=[_row_spec(512), _full_spec((2, 512)),
                  _full_spec((1, 512)), _full_spec((1, 512)),
                  _full_spec((512, 256)), _full_spec((1, 256))],
        out_specs=[_row_spec(256), _full_spec((2, 256))],
        out_shape=[jax.ShapeDtypeStruct((_B, 256), f32),
                   jax.ShapeDtypeStruct((2, 256), f32)],
    )(y1, s1, g1.reshape(1, -1), be1.reshape(1, -1), W2.T,
      b2.reshape(1, -1))

    y3, s3 = pl.pallas_call(
        _mid_body,
        grid=(_NBLK,),
        in_specs=[_row_spec(256), _full_spec((2, 256)),
                  _full_spec((1, 256)), _full_spec((1, 256)),
                  _full_spec((256, _D)), _full_spec((1, _D))],
        out_specs=[_row_spec(_D), _full_spec((2, _D))],
        out_shape=[jax.ShapeDtypeStruct((_B, _D), f32),
                   jax.ShapeDtypeStruct((2, _D), f32)],
    )(y2, s2, g2.reshape(1, -1), be2.reshape(1, -1), W3.T,
      b3.reshape(1, -1))

    out = pl.pallas_call(
        _p4_body,
        grid=(_NBLK,),
        in_specs=[_row_spec(_D), _full_spec((2, _D)),
                  _full_spec((1, _D)), _full_spec((1, _D)),
                  _full_spec((1, _D)), _full_spec((1, 1)), _vec_spec()],
        out_specs=_vec_spec(),
        out_shape=jax.ShapeDtypeStruct((_B,), f32),
    )(y3, s3, g3.reshape(1, -1), be3.reshape(1, -1), Wf,
      bf.reshape(1, 1), res)

    return out


# speedup vs baseline: 1.3977x; 1.1781x over previous
import functools

import jax
import jax.numpy as jnp
from jax import lax
from jax.experimental import pallas as pl
from jax.experimental.pallas import tpu as pltpu
from jax.experimental.pallas import tpu_sc as plsc

_B = 16384
_D = 128
_NC = 2
_NS = 16
_NW = _NC * _NS
_BPW = _B // _NW
_CH = _BPW // 128
_EPS = 1e-5

_BLK = 1024
_NBLK = _B // _BLK


def _sc_gather_body(uidx_hbm, midx_hbm, uemb_hbm, memb_hbm, ubias_hbm,
                    mbias_hbm, ue_out, me_out, ub_out, mb_out,
                    idx_u, idx_m, rows, brows_u, brows_m, sem_e, sem_b):
    wid = lax.axis_index("s") * _NC + lax.axis_index("c")
    base = wid * _BPW
    pltpu.sync_copy(uidx_hbm.at[wid], idx_u)
    pltpu.sync_copy(midx_hbm.at[wid], idx_m)
    hb = []
    for j in range(_CH):
        hb.append(pltpu.async_copy(ubias_hbm.at[idx_u.at[j]],
                                   brows_u.at[pl.ds(j * 128, 128)], sem_b))
        hb.append(pltpu.async_copy(mbias_hbm.at[idx_m.at[j]],
                                   brows_m.at[pl.ds(j * 128, 128)], sem_b))
    he = [pltpu.async_copy(uemb_hbm.at[idx_u.at[j]],
                           rows.at[pl.ds(j * 128, 128)], sem_e)
          for j in range(_CH)]
    for h in he:
        h.wait()
    pltpu.sync_copy(rows, ue_out.at[pl.ds(base, _BPW)])
    he = [pltpu.async_copy(memb_hbm.at[idx_m.at[j]],
                           rows.at[pl.ds(j * 128, 128)], sem_e)
          for j in range(_CH)]
    for h in hb:
        h.wait()
    pltpu.sync_copy(brows_u, ub_out.at[pl.ds(base, _BPW)])
    pltpu.sync_copy(brows_m, mb_out.at[pl.ds(base, _BPW)])
    for h in he:
        h.wait()
    pltpu.sync_copy(rows, me_out.at[pl.ds(base, _BPW)])


def _sc_gather(uidx, midx, uemb, memb, ubias, mbias):
    mesh = plsc.VectorSubcoreMesh(core_axis_name="c", subcore_axis_name="s")
    fn = pl.kernel(
        _sc_gather_body,
        mesh=mesh,
        out_type=(
            jax.ShapeDtypeStruct((_B, _D), jnp.float32),
            jax.ShapeDtypeStruct((_B, _D), jnp.float32),
            jax.ShapeDtypeStruct((_B,), jnp.float32),
            jax.ShapeDtypeStruct((_B,), jnp.float32),
        ),
        scratch_types=[
            pltpu.VMEM((_CH, 128), jnp.int32),
            pltpu.VMEM((_CH, 128), jnp.int32),
            pltpu.VMEM((_BPW, _D), jnp.float32),
            pltpu.VMEM((_BPW,), jnp.float32),
            pltpu.VMEM((_BPW,), jnp.float32),
            pltpu.SemaphoreType.DMA,
            pltpu.SemaphoreType.DMA,
        ],
    )
    return fn(uidx, midx, uemb, memb, ubias, mbias)


def _affine(s_sum, s_sq, g, be):
    mu = s_sum * (1.0 / _B)
    var = s_sq * (1.0 / _B) - mu * mu
    a = g * lax.rsqrt(var + _EPS)
    return a, be - mu * a


def _colstats(y, ones_row):
    s = jnp.dot(ones_row, y, preferred_element_type=jnp.float32)
    sq = jnp.dot(ones_row, y * y, preferred_element_type=jnp.float32)
    return jnp.concatenate([s, sq], axis=0)


def _acc(ref, i, val):
    @pl.when(i == 0)
    def _():
        ref[...] = val

    @pl.when(i > 0)
    def _():
        ref[...] += val


def _fused_body(ue_ref, me_ref, gu_ref, bu_ref, gm_ref, bm_ref,
                w1t_ref, b1_ref, wr_ref, sc_ref, ub_ref, mb_ref,
                g1_ref, be1_ref, w2t_ref, b2_ref,
                g2_ref, be2_ref, w3t_ref, b3_ref,
                g3_ref, be3_ref, wf_ref, bf_ref,
                out_ref,
                y1_s, y2_s, y3_s, res_s, s0_s, s1_s, s2_s, s3_s):
    p = pl.program_id(0)
    i = pl.program_id(1)
    rows = pl.ds(i * _BLK, _BLK)
    ones_row = jnp.ones((1, _BLK), jnp.float32)

    @pl.when(p == 0)
    def _phase0():
        st = jnp.concatenate([_colstats(ue_ref[...], ones_row),
                              _colstats(me_ref[...], ones_row)], axis=0)
        _acc(s0_s, i, st)

    @pl.when(p == 1)
    def _phase1():
        s0 = s0_s[...]
        au, cu = _affine(s0[0:1], s0[1:2], gu_ref[...], bu_ref[...])
        am, cm = _affine(s0[2:3], s0[3:4], gm_ref[...], bm_ref[...])
        comb = jnp.concatenate([ue_ref[...] * au + cu,
                                me_ref[...] * am + cm], axis=1)
        y1 = jnp.dot(comb, w1t_ref[...],
                     preferred_element_type=jnp.float32) + b1_ref[...]
        y1_s[rows, :] = y1
        res_s[rows] = (jnp.sum(comb * wr_ref[...], axis=1)
                       + sc_ref[0, 0] + ub_ref[...] + mb_ref[...])
        _acc(s1_s, i, _colstats(y1, ones_row))

    @pl.when(p == 2)
    def _phase2():
        s1 = s1_s[...]
        a, c = _affine(s1[0:1], s1[1:2], g1_ref[...], be1_ref[...])
        x = jnp.maximum(y1_s[rows, :] * a + c, 0.0)
        y2 = jnp.dot(x, w2t_ref[...],
                     preferred_element_type=jnp.float32) + b2_ref[...]
        y2_s[rows, :] = y2
        _acc(s2_s, i, _colstats(y2, ones_row))

    @pl.when(p == 3)
    def _phase3():
        s2 = s2_s[...]
        a, c = _affine(s2[0:1], s2[1:2], g2_ref[...], be2_ref[...])
        x = jnp.maximum(y2_s[rows, :] * a + c, 0.0)
        y3 = jnp.dot(x, w3t_ref[...],
                     preferred_element_type=jnp.float32) + b3_ref[...]
        y3_s[rows, :] = y3
        _acc(s3_s, i, _colstats(y3, ones_row))

    @pl.when(p == 4)
    def _phase4():
        s3 = s3_s[...]
        a, c = _affine(s3[0:1], s3[1:2], g3_ref[...], be3_ref[...])
        x = jnp.maximum(y3_s[rows, :] * a + c, 0.0)
        out_ref[...] = (jnp.sum(x * wf_ref[...], axis=1) + bf_ref[0, 0]
                        + res_s[rows])


def _emb_spec():
    return pl.BlockSpec((_BLK, _D), lambda p, i: (jnp.where(p <= 1, i, 0), 0))


def _bias_spec():
    return pl.BlockSpec((_BLK,), lambda p, i: (jnp.where(p == 1, i, 0),))


def _full_spec(shape):
    nd = len(shape)
    return pl.BlockSpec(shape, lambda p, i: (0,) * nd)


def kernel(users, movies, user_emb, movie_emb, user_bias_t, movie_bias_t,
           global_bias, gu, bu, gm, bm, W1, b1, g1, be1, W2, b2, g2, be2,
           W3, b3, g3, be3, Wf, bf, Wr, br):
    uidx = users.astype(jnp.int32).reshape(_NW, _CH, 128)
    midx = movies.astype(jnp.int32).reshape(_NW, _CH, 128)
    ue, me, ubg, mbg = _sc_gather(uidx, midx, user_emb, movie_emb,
                                  user_bias_t.reshape(-1),
                                  movie_bias_t.reshape(-1))

    f32 = jnp.float32
    scalar_c = (global_bias + br).reshape(1, 1)
    out = pl.pallas_call(
        _fused_body,
        grid=(5, _NBLK),
        in_specs=[_emb_spec(), _emb_spec(),
                  _full_spec((1, _D)), _full_spec((1, _D)),
                  _full_spec((1, _D)), _full_spec((1, _D)),
                  _full_spec((2 * _D, 512)), _full_spec((1, 512)),
                  _full_spec((1, 2 * _D)), _full_spec((1, 1)),
                  _bias_spec(), _bias_spec(),
                  _full_spec((1, 512)), _full_spec((1, 512)),
                  _full_spec((512, 256)), _full_spec((1, 256)),
                  _full_spec((1, 256)), _full_spec((1, 256)),
                  _full_spec((256, _D)), _full_spec((1, _D)),
                  _full_spec((1, _D)), _full_spec((1, _D)),
                  _full_spec((1, _D)), _full_spec((1, 1))],
        out_specs=pl.BlockSpec((_BLK,), lambda p, i: (jnp.where(p == 4, i, 0),)),
        out_shape=jax.ShapeDtypeStruct((_B,), f32),
        scratch_shapes=[
            pltpu.VMEM((_B, 512), f32),
            pltpu.VMEM((_B, 256), f32),
            pltpu.VMEM((_B, _D), f32),
            pltpu.VMEM((_B,), f32),
            pltpu.VMEM((4, _D), f32),
            pltpu.VMEM((2, 512), f32),
            pltpu.VMEM((2, 256), f32),
            pltpu.VMEM((2, _D), f32),
        ],
        compiler_params=pltpu.CompilerParams(
            vmem_limit_bytes=100 * 1024 * 1024,
        ),
    )(ue, me, gu.reshape(1, -1), bu.reshape(1, -1),
      gm.reshape(1, -1), bm.reshape(1, -1), W1.T, b1.reshape(1, -1),
      Wr, scalar_c, ubg, mbg,
      g1.reshape(1, -1), be1.reshape(1, -1), W2.T, b2.reshape(1, -1),
      g2.reshape(1, -1), be2.reshape(1, -1), W3.T, b3.reshape(1, -1),
      g3.reshape(1, -1), be3.reshape(1, -1), Wf, bf.reshape(1, 1))

    return out
